# baseline (speedup 1.0000x reference)
import jax
import jax.numpy as jnp
from jax import lax
from jax.experimental import pallas as pl
from jax.experimental.pallas import tpu as pltpu
from jax.experimental.pallas import tpu_sc as plsc

_NUM_FIELDS = 26
_NPAIR = _NUM_FIELDS // 2
_VOCAB = 100000
_D = 64
_DP = 128
_B = 16384
_NC = 2
_NS = 16
_NW = _NC * _NS
_RPW = _B // _NW
_HALF = _RPW // 2
_CHUNK = 128
_NCHUNK = _RPW // _CHUNK
_LANES = 16
_VPR = _D // _LANES

_CV = 512
_NBV = (_VOCAB + _CV - 1) // _CV


def _tc_pack_body(in0_ref, in1_ref, o_ref):
    t0 = jnp.transpose(in0_ref[0], (1, 0))
    t1 = jnp.transpose(in1_ref[0], (1, 0))
    o_ref[0] = jnp.concatenate([t0, t1], axis=1)


def _body(xt_hbm, tab_hbm, out_hbm, idx_v, buf_v, acc_v, sem0, sem1):
    wid = lax.axis_index("s") * _NC + lax.axis_index("c")
    base = wid * _RPW

    pltpu.sync_copy(xt_hbm.at[:, wid], idx_v)

    for f in range(2, _NUM_FIELDS):
        off = jnp.full((_LANES,), (f // 2) * _VOCAB, dtype=jnp.int32)

        def _off_body(c, _, f=f, off=off):
            for v in range(_CHUNK // _LANES):
                plsc.addupdate(idx_v.at[f, c, pl.ds(v * _LANES, _LANES)], off)
            return 0

        lax.fori_loop(0, _NCHUNK, _off_body, 0)

    sems = (sem0, sem1)
    nch = _HALF // _CHUNK

    for h in range(2):
        def _fire(f, h=h):
            p = f % 2
            return [
                pltpu.async_copy(
                    tab_hbm.at[idx_v.at[f, h * nch + c]],
                    buf_v.at[p, pl.ds(c * _CHUNK, _CHUNK)],
                    sems[p],
                )
                for c in range(nch)
            ]

        handles = _fire(0)
        for f in range(_NUM_FIELDS):
            p = f % 2
            col0 = (f % 2) * _D
            nxt = _fire(f + 1) if f + 1 < _NUM_FIELDS else None
            for hd in handles:
                hd.wait()
            handles = nxt

            if f == 0:
                def _init_body(r, _, p=p):
                    for v in range(_VPR):
                        acc_v[r, pl.ds(v * _LANES, _LANES)] = buf_v[
                            p, r, pl.ds(v * _LANES, _LANES)
                        ]
                    return 0

                lax.fori_loop(0, _HALF, _init_body, 0)
            else:
                def _acc_body(r, _, p=p, col0=col0):
                    for v in range(_VPR):
                        plsc.addupdate(
                            acc_v.at[r, pl.ds(v * _LANES, _LANES)],
                            buf_v[p, r, pl.ds(col0 + v * _LANES, _LANES)],
                        )
                    return 0

                lax.fori_loop(0, _HALF, _acc_body, 0)

        pltpu.sync_copy(acc_v, out_hbm.at[pl.ds(base + h * _HALF, _HALF)])


@jax.jit
def _run(xt4, tabT):
    packed = pl.pallas_call(
        _tc_pack_body,
        grid=(_NPAIR, _NBV),
        in_specs=[
            pl.BlockSpec((1, _D, _CV), lambda i, j: (2 * i, 0, j)),
            pl.BlockSpec((1, _D, _CV), lambda i, j: (2 * i + 1, 0, j)),
        ],
        out_specs=pl.BlockSpec((1, _CV, _DP), lambda i, j: (i, j, 0)),
        out_shape=jax.ShapeDtypeStruct((_NPAIR, _VOCAB, _DP), jnp.float32),
    )(tabT, tabT)
    flat_tab = packed.reshape(_NPAIR * _VOCAB, _DP)

    mesh = plsc.VectorSubcoreMesh(core_axis_name="c", subcore_axis_name="s")
    f = pl.kernel(
        _body,
        out_type=jax.ShapeDtypeStruct((_B, _D), jnp.float32),
        mesh=mesh,
        compiler_params=pltpu.CompilerParams(use_tc_tiling_on_sc=False),
        scratch_types=[
            pltpu.VMEM((_NUM_FIELDS, _NCHUNK, _CHUNK), jnp.int32),
            pltpu.VMEM((2, _HALF, _DP), jnp.float32),
            pltpu.VMEM((_HALF, _D), jnp.float32),
            pltpu.SemaphoreType.DMA,
            pltpu.SemaphoreType.DMA,
        ],
    )
    return f(xt4, flat_tab)


def kernel(x_cat, tables):
    xt4 = x_cat.T.reshape(_NUM_FIELDS, _NW, _NCHUNK, _CHUNK)
    tabT = tables.transpose(0, 2, 1)
    return _run(xt4, tabT)


# speedup vs baseline: 1.8902x; 1.8902x over previous
import jax
import jax.numpy as jnp
from jax import lax
from jax.experimental import pallas as pl
from jax.experimental.pallas import tpu as pltpu
from jax.experimental.pallas import tpu_sc as plsc

_NUM_FIELDS = 26
_NPAIR = _NUM_FIELDS // 2
_VOCAB = 100000
_D = 64
_DP = 128
_B = 16384
_NC = 2
_NS = 16
_NW = _NC * _NS
_RPW = _B // _NW
_HALF = _RPW // 2
_CHUNK = 128
_NCHUNK = _RPW // _CHUNK
_LANES = 16
_VPR = _D // _LANES

_CV = 2048
_NBV = (_VOCAB + _CV - 1) // _CV


def _tc_pack_body(in0_ref, in1_ref, o_ref):
    ident = (
        lax.broadcasted_iota(jnp.int32, (_D, _D), 0)
        == lax.broadcasted_iota(jnp.int32, (_D, _D), 1)
    ).astype(jnp.float32)
    dn = (((0,), (0,)), ((), ()))
    t0 = lax.dot_general(
        in0_ref[0], ident, dn, preferred_element_type=jnp.float32
    )
    t1 = lax.dot_general(
        in1_ref[0], ident, dn, preferred_element_type=jnp.float32
    )
    o_ref[0] = jnp.concatenate([t0, t1], axis=1)


def _body(xt_hbm, tab_hbm, out_hbm, idx_v, buf_v, acc_v, sem0, sem1):
    wid = lax.axis_index("s") * _NC + lax.axis_index("c")
    base = wid * _RPW

    pltpu.sync_copy(xt_hbm.at[:, wid], idx_v)

    for f in range(2, _NUM_FIELDS):
        off = jnp.full((_LANES,), (f // 2) * _VOCAB, dtype=jnp.int32)

        def _off_body(c, _, f=f, off=off):
            for v in range(_CHUNK // _LANES):
                plsc.addupdate(idx_v.at[f, c, pl.ds(v * _LANES, _LANES)], off)
            return 0

        lax.fori_loop(0, _NCHUNK, _off_body, 0)

    sems = (sem0, sem1)
    nch = _HALF // _CHUNK

    for h in range(2):
        def _fire(f, h=h):
            p = f % 2
            return [
                pltpu.async_copy(
                    tab_hbm.at[idx_v.at[f, h * nch + c]],
                    buf_v.at[p, pl.ds(c * _CHUNK, _CHUNK)],
                    sems[p],
                )
                for c in range(nch)
            ]

        handles = _fire(0)
        for f in range(_NUM_FIELDS):
            p = f % 2
            col0 = (f % 2) * _D
            nxt = _fire(f + 1) if f + 1 < _NUM_FIELDS else None
            for hd in handles:
                hd.wait()
            handles = nxt

            if f == 0:
                def _init_body(r, _, p=p):
                    for v in range(_VPR):
                        acc_v[r, pl.ds(v * _LANES, _LANES)] = buf_v[
                            p, r, pl.ds(v * _LANES, _LANES)
                        ]
                    return 0

                lax.fori_loop(0, _HALF, _init_body, 0)
            else:
                def _acc_body(r, _, p=p, col0=col0):
                    for v in range(_VPR):
                        plsc.addupdate(
                            acc_v.at[r, pl.ds(v * _LANES, _LANES)],
                            buf_v[p, r, pl.ds(col0 + v * _LANES, _LANES)],
                        )
                    return 0

                lax.fori_loop(0, _HALF, _acc_body, 0)

        pltpu.sync_copy(acc_v, out_hbm.at[pl.ds(base + h * _HALF, _HALF)])


@jax.jit
def _run(xt4, tabT):
    packed = pl.pallas_call(
        _tc_pack_body,
        grid=(_NPAIR, _NBV),
        in_specs=[
            pl.BlockSpec((1, _D, _CV), lambda i, j: (2 * i, 0, j)),
            pl.BlockSpec((1, _D, _CV), lambda i, j: (2 * i + 1, 0, j)),
        ],
        out_specs=pl.BlockSpec((1, _CV, _DP), lambda i, j: (i, j, 0)),
        out_shape=jax.ShapeDtypeStruct((_NPAIR, _VOCAB, _DP), jnp.float32),
    )(tabT, tabT)
    flat_tab = packed.reshape(_NPAIR * _VOCAB, _DP)

    mesh = plsc.VectorSubcoreMesh(core_axis_name="c", subcore_axis_name="s")
    f = pl.kernel(
        _body,
        out_type=jax.ShapeDtypeStruct((_B, _D), jnp.float32),
        mesh=mesh,
        compiler_params=pltpu.CompilerParams(use_tc_tiling_on_sc=False),
        scratch_types=[
            pltpu.VMEM((_NUM_FIELDS, _NCHUNK, _CHUNK), jnp.int32),
            pltpu.VMEM((2, _HALF, _DP), jnp.float32),
            pltpu.VMEM((_HALF, _D), jnp.float32),
            pltpu.SemaphoreType.DMA,
            pltpu.SemaphoreType.DMA,
        ],
    )
    return f(xt4, flat_tab)


def kernel(x_cat, tables):
    xt4 = x_cat.T.reshape(_NUM_FIELDS, _NW, _NCHUNK, _CHUNK)
    tabT = tables.transpose(0, 2, 1)
    return _run(xt4, tabT)


# speedup vs baseline: 1.8920x; 1.0010x over previous
import jax
import jax.numpy as jnp
from jax import lax
from jax.experimental import pallas as pl
from jax.experimental.pallas import tpu as pltpu
from jax.experimental.pallas import tpu_sc as plsc

_NUM_FIELDS = 26
_NPAIR = _NUM_FIELDS // 2
_VOCAB = 100000
_D = 64
_DP = 128
_B = 16384
_NC = 2
_NS = 16
_NW = _NC * _NS
_RPW = _B // _NW
_HALF = _RPW // 2
_CHUNK = 128
_NCHUNK = _RPW // _CHUNK
_LANES = 16
_VPR = _D // _LANES

_CV = 2048
_NBV = (_VOCAB + _CV - 1) // _CV


def _tc_pack_body(in0_ref, in1_ref, o_ref):
    t0 = jnp.transpose(in0_ref[0], (1, 0))
    t1 = jnp.transpose(in1_ref[0], (1, 0))
    o_ref[0] = jnp.concatenate([t0, t1], axis=1)


def _body(xt_hbm, tab_hbm, out_hbm, idx_v, buf_v, acc_v, sem0, sem1):
    wid = lax.axis_index("s") * _NC + lax.axis_index("c")
    base = wid * _RPW

    pltpu.sync_copy(xt_hbm.at[:, wid], idx_v)

    for f in range(2, _NUM_FIELDS):
        off = jnp.full((_LANES,), (f // 2) * _VOCAB, dtype=jnp.int32)

        def _off_body(c, _, f=f, off=off):
            for v in range(_CHUNK // _LANES):
                plsc.addupdate(idx_v.at[f, c, pl.ds(v * _LANES, _LANES)], off)
            return 0

        lax.fori_loop(0, _NCHUNK, _off_body, 0)

    sems = (sem0, sem1)
    nch = _HALF // _CHUNK

    for h in range(2):
        def _fire(f, h=h):
            p = f % 2
            return [
                pltpu.async_copy(
                    tab_hbm.at[idx_v.at[f, h * nch + c]],
                    buf_v.at[p, pl.ds(c * _CHUNK, _CHUNK)],
                    sems[p],
                )
                for c in range(nch)
            ]

        handles = _fire(0)
        for f in range(_NUM_FIELDS):
            p = f % 2
            col0 = (f % 2) * _D
            nxt = _fire(f + 1) if f + 1 < _NUM_FIELDS else None
            for hd in handles:
                hd.wait()
            handles = nxt

            if f == 0:
                def _init_body(r, _, p=p):
                    for v in range(_VPR):
                        acc_v[r, pl.ds(v * _LANES, _LANES)] = buf_v[
                            p, r, pl.ds(v * _LANES, _LANES)
                        ]
                    return 0

                lax.fori_loop(0, _HALF, _init_body, 0)
            else:
                def _acc_body(r, _, p=p, col0=col0):
                    for v in range(_VPR):
                        plsc.addupdate(
                            acc_v.at[r, pl.ds(v * _LANES, _LANES)],
                            buf_v[p, r, pl.ds(col0 + v * _LANES, _LANES)],
                        )
                    return 0

                lax.fori_loop(0, _HALF, _acc_body, 0)

        pltpu.sync_copy(acc_v, out_hbm.at[pl.ds(base + h * _HALF, _HALF)])


@jax.jit
def _run(xt4, tabT):
    packed = pl.pallas_call(
        _tc_pack_body,
        grid=(_NPAIR, _NBV),
        in_specs=[
            pl.BlockSpec((1, _D, _CV), lambda i, j: (2 * i, 0, j)),
            pl.BlockSpec((1, _D, _CV), lambda i, j: (2 * i + 1, 0, j)),
        ],
        out_specs=pl.BlockSpec((1, _CV, _DP), lambda i, j: (i, j, 0)),
        out_shape=jax.ShapeDtypeStruct((_NPAIR, _VOCAB, _DP), jnp.float32),
    )(tabT, tabT)
    flat_tab = packed.reshape(_NPAIR * _VOCAB, _DP)

    mesh = plsc.VectorSubcoreMesh(core_axis_name="c", subcore_axis_name="s")
    f = pl.kernel(
        _body,
        out_type=jax.ShapeDtypeStruct((_B, _D), jnp.float32),
        mesh=mesh,
        compiler_params=pltpu.CompilerParams(use_tc_tiling_on_sc=False),
        scratch_types=[
            pltpu.VMEM((_NUM_FIELDS, _NCHUNK, _CHUNK), jnp.int32),
            pltpu.VMEM((2, _HALF, _DP), jnp.float32),
            pltpu.VMEM((_HALF, _D), jnp.float32),
            pltpu.SemaphoreType.DMA,
            pltpu.SemaphoreType.DMA,
        ],
    )
    return f(xt4, flat_tab)


def kernel(x_cat, tables):
    xt4 = x_cat.T.reshape(_NUM_FIELDS, _NW, _NCHUNK, _CHUNK)
    tabT = tables.transpose(0, 2, 1)
    return _run(xt4, tabT)


# speedup vs baseline: 2.2378x; 1.1828x over previous
import jax
import jax.numpy as jnp
from jax import lax
from jax.experimental import pallas as pl
from jax.experimental.pallas import tpu as pltpu
from jax.experimental.pallas import tpu_sc as plsc

_NUM_FIELDS = 26
_NPAIR = _NUM_FIELDS // 2
_VOCAB = 100000
_D = 64
_DP = 128
_B = 16384
_NC = 2
_NS = 16
_NW = _NC * _NS
_RPW = _B // _NW
_HALF = _RPW // 2
_CHUNK = 128
_NCHUNK = _RPW // _CHUNK
_LANES = 16
_VPR = _D // _LANES

_CV = 4096
_NBV = (_VOCAB + _CV - 1) // _CV


def _tc_pack_body(in0_ref, in1_ref, o_ref):
    t0 = jnp.transpose(in0_ref[0], (1, 0))
    t1 = jnp.transpose(in1_ref[0], (1, 0))
    o_ref[0] = jnp.concatenate([t0, t1], axis=1)


def _body(xt_hbm, tab_hbm, out_hbm, idx_v, buf_v, acc_v, sem0, sem1):
    wid = lax.axis_index("s") * _NC + lax.axis_index("c")
    base = wid * _RPW

    pltpu.sync_copy(xt_hbm.at[:, wid], idx_v)

    for f in range(2, _NUM_FIELDS):
        off = jnp.full((_LANES,), (f // 2) * _VOCAB, dtype=jnp.int32)

        def _off_body(c, _, f=f, off=off):
            for v in range(_CHUNK // _LANES):
                plsc.addupdate(idx_v.at[f, c, pl.ds(v * _LANES, _LANES)], off)
            return 0

        lax.fori_loop(0, _NCHUNK, _off_body, 0)

    sems = (sem0, sem1)
    nch = _HALF // _CHUNK

    for h in range(2):
        def _fire(f, h=h):
            p = f % 2
            return [
                pltpu.async_copy(
                    tab_hbm.at[idx_v.at[f, h * nch + c]],
                    buf_v.at[p, pl.ds(c * _CHUNK, _CHUNK)],
                    sems[p],
                )
                for c in range(nch)
            ]

        handles = _fire(0)
        for f in range(_NUM_FIELDS):
            p = f % 2
            col0 = (f % 2) * _D
            nxt = _fire(f + 1) if f + 1 < _NUM_FIELDS else None
            for hd in handles:
                hd.wait()
            handles = nxt

            if f == 0:
                def _init_body(r, _, p=p):
                    for v in range(_VPR):
                        acc_v[r, pl.ds(v * _LANES, _LANES)] = buf_v[
                            p, r, pl.ds(v * _LANES, _LANES)
                        ]
                    return 0

                lax.fori_loop(0, _HALF, _init_body, 0)
            else:
                def _acc_body(r, _, p=p, col0=col0):
                    for v in range(_VPR):
                        plsc.addupdate(
                            acc_v.at[r, pl.ds(v * _LANES, _LANES)],
                            buf_v[p, r, pl.ds(col0 + v * _LANES, _LANES)],
                        )
                    return 0

                lax.fori_loop(0, _HALF, _acc_body, 0)

        pltpu.sync_copy(acc_v, out_hbm.at[pl.ds(base + h * _HALF, _HALF)])


@jax.jit
def _run(xt4, tabT):
    packed = pl.pallas_call(
        _tc_pack_body,
        grid=(_NPAIR, _NBV),
        in_specs=[
            pl.BlockSpec((1, _D, _CV), lambda i, j: (2 * i, 0, j)),
            pl.BlockSpec((1, _D, _CV), lambda i, j: (2 * i + 1, 0, j)),
        ],
        out_specs=pl.BlockSpec((1, _CV, _DP), lambda i, j: (i, j, 0)),
        out_shape=jax.ShapeDtypeStruct((_NPAIR, _VOCAB, _DP), jnp.float32),
    )(tabT, tabT)
    flat_tab = packed.reshape(_NPAIR * _VOCAB, _DP)

    mesh = plsc.VectorSubcoreMesh(core_axis_name="c", subcore_axis_name="s")
    f = pl.kernel(
        _body,
        out_type=jax.ShapeDtypeStruct((_B, _D), jnp.float32),
        mesh=mesh,
        compiler_params=pltpu.CompilerParams(use_tc_tiling_on_sc=False),
        scratch_types=[
            pltpu.VMEM((_NUM_FIELDS, _NCHUNK, _CHUNK), jnp.int32),
            pltpu.VMEM((2, _HALF, _DP), jnp.float32),
            pltpu.VMEM((_HALF, _D), jnp.float32),
            pltpu.SemaphoreType.DMA,
            pltpu.SemaphoreType.DMA,
        ],
    )
    return f(xt4, flat_tab)


def kernel(x_cat, tables):
    xt4 = x_cat.T.reshape(_NUM_FIELDS, _NW, _NCHUNK, _CHUNK)
    tabT = tables.transpose(0, 2, 1)
    return _run(xt4, tabT)


# speedup vs baseline: 2.4285x; 1.0852x over previous
import jax
import jax.numpy as jnp
from jax import lax
from jax.experimental import pallas as pl
from jax.experimental.pallas import tpu as pltpu
from jax.experimental.pallas import tpu_sc as plsc

_NUM_FIELDS = 26
_NPAIR = _NUM_FIELDS // 2
_VOCAB = 100000
_D = 64
_DP = 128
_B = 16384
_NC = 2
_NS = 16
_NW = _NC * _NS
_RPW = _B // _NW
_HALF = _RPW // 2
_CHUNK = 128
_NCHUNK = _RPW // _CHUNK
_LANES = 16
_VPR = _D // _LANES

_CV = 8192
_NBV = (_VOCAB + _CV - 1) // _CV


def _tc_pack_body(in0_ref, in1_ref, o_ref):
    t0 = jnp.transpose(in0_ref[0], (1, 0))
    t1 = jnp.transpose(in1_ref[0], (1, 0))
    o_ref[0] = jnp.concatenate([t0, t1], axis=1)


def _body(xt_hbm, tab_hbm, out_hbm, idx_v, buf_v, acc_v, sem0, sem1):
    wid = lax.axis_index("s") * _NC + lax.axis_index("c")
    base = wid * _RPW

    pltpu.sync_copy(xt_hbm.at[:, wid], idx_v)

    for f in range(2, _NUM_FIELDS):
        off = jnp.full((_LANES,), (f // 2) * _VOCAB, dtype=jnp.int32)

        def _off_body(c, _, f=f, off=off):
            for v in range(_CHUNK // _LANES):
                plsc.addupdate(idx_v.at[f, c, pl.ds(v * _LANES, _LANES)], off)
            return 0

        lax.fori_loop(0, _NCHUNK, _off_body, 0)

    sems = (sem0, sem1)
    nch = _HALF // _CHUNK

    for h in range(2):
        def _fire(f, h=h):
            p = f % 2
            return [
                pltpu.async_copy(
                    tab_hbm.at[idx_v.at[f, h * nch + c]],
                    buf_v.at[p, pl.ds(c * _CHUNK, _CHUNK)],
                    sems[p],
                )
                for c in range(nch)
            ]

        handles = _fire(0)
        for f in range(_NUM_FIELDS):
            p = f % 2
            col0 = (f % 2) * _D
            nxt = _fire(f + 1) if f + 1 < _NUM_FIELDS else None
            for hd in handles:
                hd.wait()
            handles = nxt

            if f == 0:
                def _init_body(r, _, p=p):
                    for v in range(_VPR):
                        acc_v[r, pl.ds(v * _LANES, _LANES)] = buf_v[
                            p, r, pl.ds(v * _LANES, _LANES)
                        ]
                    return 0

                lax.fori_loop(0, _HALF, _init_body, 0)
            else:
                def _acc_body(r, _, p=p, col0=col0):
                    for v in range(_VPR):
                        plsc.addupdate(
                            acc_v.at[r, pl.ds(v * _LANES, _LANES)],
                            buf_v[p, r, pl.ds(col0 + v * _LANES, _LANES)],
                        )
                    return 0

                lax.fori_loop(0, _HALF, _acc_body, 0)

        pltpu.sync_copy(acc_v, out_hbm.at[pl.ds(base + h * _HALF, _HALF)])


@jax.jit
def _run(xt4, tabT):
    packed = pl.pallas_call(
        _tc_pack_body,
        grid=(_NPAIR, _NBV),
        in_specs=[
            pl.BlockSpec((1, _D, _CV), lambda i, j: (2 * i, 0, j)),
            pl.BlockSpec((1, _D, _CV), lambda i, j: (2 * i + 1, 0, j)),
        ],
        out_specs=pl.BlockSpec((1, _CV, _DP), lambda i, j: (i, j, 0)),
        out_shape=jax.ShapeDtypeStruct((_NPAIR, _VOCAB, _DP), jnp.float32),
    )(tabT, tabT)
    flat_tab = packed.reshape(_NPAIR * _VOCAB, _DP)

    mesh = plsc.VectorSubcoreMesh(core_axis_name="c", subcore_axis_name="s")
    f = pl.kernel(
        _body,
        out_type=jax.ShapeDtypeStruct((_B, _D), jnp.float32),
        mesh=mesh,
        compiler_params=pltpu.CompilerParams(use_tc_tiling_on_sc=False),
        scratch_types=[
            pltpu.VMEM((_NUM_FIELDS, _NCHUNK, _CHUNK), jnp.int32),
            pltpu.VMEM((2, _HALF, _DP), jnp.float32),
            pltpu.VMEM((_HALF, _D), jnp.float32),
            pltpu.SemaphoreType.DMA,
            pltpu.SemaphoreType.DMA,
        ],
    )
    return f(xt4, flat_tab)


def kernel(x_cat, tables):
    xt4 = x_cat.T.reshape(_NUM_FIELDS, _NW, _NCHUNK, _CHUNK)
    tabT = tables.transpose(0, 2, 1)
    return _run(xt4, tabT)


# speedup vs baseline: 2.4307x; 1.0009x over previous
import jax
import jax.numpy as jnp
from jax import lax
from jax.experimental import pallas as pl
from jax.experimental.pallas import tpu as pltpu
from jax.experimental.pallas import tpu_sc as plsc

_NUM_FIELDS = 26
_NPAIR = _NUM_FIELDS // 2
_VOCAB = 100000
_D = 64
_DP = 128
_B = 16384
_NC = 2
_NS = 16
_NW = _NC * _NS
_RPW = _B // _NW
_HALF = _RPW // 2
_CHUNK = 128
_NCHUNK = _RPW // _CHUNK
_LANES = 16
_VPR = _D // _LANES

_CV = 16384
_NBV = (_VOCAB + _CV - 1) // _CV


def _tc_pack_body(in0_ref, in1_ref, o_ref):
    t0 = jnp.transpose(in0_ref[0], (1, 0))
    t1 = jnp.transpose(in1_ref[0], (1, 0))
    o_ref[0] = jnp.concatenate([t0, t1], axis=1)


def _body(xt_hbm, tab_hbm, out_hbm, idx_v, buf_v, acc_v, sem0, sem1):
    wid = lax.axis_index("s") * _NC + lax.axis_index("c")
    base = wid * _RPW

    pltpu.sync_copy(xt_hbm.at[:, wid], idx_v)

    for f in range(2, _NUM_FIELDS):
        off = jnp.full((_LANES,), (f // 2) * _VOCAB, dtype=jnp.int32)

        def _off_body(c, _, f=f, off=off):
            for v in range(_CHUNK // _LANES):
                plsc.addupdate(idx_v.at[f, c, pl.ds(v * _LANES, _LANES)], off)
            return 0

        lax.fori_loop(0, _NCHUNK, _off_body, 0)

    sems = (sem0, sem1)
    nch = _HALF // _CHUNK

    for h in range(2):
        def _fire(f, h=h):
            p = f % 2
            return [
                pltpu.async_copy(
                    tab_hbm.at[idx_v.at[f, h * nch + c]],
                    buf_v.at[p, pl.ds(c * _CHUNK, _CHUNK)],
                    sems[p],
                )
                for c in range(nch)
            ]

        handles = _fire(0)
        for f in range(_NUM_FIELDS):
            p = f % 2
            col0 = (f % 2) * _D
            nxt = _fire(f + 1) if f + 1 < _NUM_FIELDS else None
            for hd in handles:
                hd.wait()
            handles = nxt

            if f == 0:
                def _init_body(r, _, p=p):
                    for v in range(_VPR):
                        acc_v[r, pl.ds(v * _LANES, _LANES)] = buf_v[
                            p, r, pl.ds(v * _LANES, _LANES)
                        ]
                    return 0

                lax.fori_loop(0, _HALF, _init_body, 0)
            else:
                def _acc_body(r, _, p=p, col0=col0):
                    for v in range(_VPR):
                        plsc.addupdate(
                            acc_v.at[r, pl.ds(v * _LANES, _LANES)],
                            buf_v[p, r, pl.ds(col0 + v * _LANES, _LANES)],
                        )
                    return 0

                lax.fori_loop(0, _HALF, _acc_body, 0)

        pltpu.sync_copy(acc_v, out_hbm.at[pl.ds(base + h * _HALF, _HALF)])


@jax.jit
def _run(xt4, tabT):
    packed = pl.pallas_call(
        _tc_pack_body,
        grid=(_NPAIR, _NBV),
        in_specs=[
            pl.BlockSpec((1, _D, _CV), lambda i, j: (2 * i, 0, j)),
            pl.BlockSpec((1, _D, _CV), lambda i, j: (2 * i + 1, 0, j)),
        ],
        out_specs=pl.BlockSpec((1, _CV, _DP), lambda i, j: (i, j, 0)),
        out_shape=jax.ShapeDtypeStruct((_NPAIR, _VOCAB, _DP), jnp.float32),
    )(tabT, tabT)
    flat_tab = packed.reshape(_NPAIR * _VOCAB, _DP)

    mesh = plsc.VectorSubcoreMesh(core_axis_name="c", subcore_axis_name="s")
    f = pl.kernel(
        _body,
        out_type=jax.ShapeDtypeStruct((_B, _D), jnp.float32),
        mesh=mesh,
        compiler_params=pltpu.CompilerParams(use_tc_tiling_on_sc=False),
        scratch_types=[
            pltpu.VMEM((_NUM_FIELDS, _NCHUNK, _CHUNK), jnp.int32),
            pltpu.VMEM((2, _HALF, _DP), jnp.float32),
            pltpu.VMEM((_HALF, _D), jnp.float32),
            pltpu.SemaphoreType.DMA,
            pltpu.SemaphoreType.DMA,
        ],
    )
    return f(xt4, flat_tab)


def kernel(x_cat, tables):
    xt4 = x_cat.T.reshape(_NUM_FIELDS, _NW, _NCHUNK, _CHUNK)
    tabT = tables.transpose(0, 2, 1)
    return _run(xt4, tabT)
